# baseline (speedup 1.0000x reference)
import functools

import jax
import jax.numpy as jnp
import numpy as np
from jax import lax
from jax.experimental import pallas as pl
from jax.experimental.pallas import tpu as pltpu
from jax.experimental.pallas import tpu_sc as plsc

N = 10000
D = 128
DH = 64
NH = 16
N_SP = 10112
ROWS_PER_TILE = N_SP // 16
E = 320000
CH = 128
SUB = CH // 128
N_CHUNKS = 161
EP_T = N_CHUNKS * CH
E_PAD = EP_T * 16
ROWS_E = EP_T // 128
E_IDX_ROWS = E_PAD // 128 + 8
R_BLK = 400
F32 = jnp.float32


R_SP = N_SP // 16


def _prep_tail(f, Wh_ref, WAs_ref, WAd_ref, Bh_ref):
    xlh = jnp.dot(f, Wh_ref[0], preferred_element_type=F32)
    at = jnp.dot(f, WAs_ref[...], preferred_element_type=F32)
    dt = jnp.dot(f, WAd_ref[...], preferred_element_type=F32)
    z = at + dt
    pd = jnp.exp(jnp.maximum(z, 0.2 * z))
    iacch = xlh * jnp.dot(pd, Bh_ref[0], preferred_element_type=F32)
    return xlh, at, dt, iacch, pd


def _pre_body(f_ref, Wh_ref, WAs_ref, WAd_ref, Bh_ref,
              xlh_ref, at_ref, dt_ref, iacch_ref, iden_ref):
    xlh, at, dt, iacch, pd = _prep_tail(f_ref[...], Wh_ref, WAs_ref,
                                        WAd_ref, Bh_ref)
    xlh_ref[...] = xlh
    at_ref[...] = at
    dt_ref[...] = dt
    iacch_ref[...] = iacch
    iden_ref[...] = pd


def _mid_body(acca_ref, accb_ref, den_ref, b_ref, B0_ref,
              Wh_ref, WAs_ref, WAd_ref, Bh_ref,
              xlh_ref, at_ref, dt_ref, iacch_ref, iden_ref):
    acc = jnp.concatenate([acca_ref[...], accb_ref[...]], axis=1)
    den = den_ref[0]
    dbc = jnp.dot(den, B0_ref[...], preferred_element_type=F32)
    h = acc / jnp.maximum(dbc, 1e-30) + b_ref[...]
    h = jnp.maximum(h, 0.0)
    s = jnp.sum(jnp.abs(h), axis=1, keepdims=True)
    h = h / jnp.maximum(s, 1e-12)
    xlh, at, dt, iacch, pd = _prep_tail(h, Wh_ref, WAs_ref, WAd_ref, Bh_ref)
    xlh_ref[...] = xlh
    at_ref[...] = at
    dt_ref[...] = dt
    iacch_ref[...] = iacch
    iden_ref[...] = pd


def _post_body(acca_ref, accb_ref, den_ref, b_ref, B1_ref, out_ref):
    acc = jnp.concatenate([acca_ref[...], accb_ref[...]], axis=1)
    den = den_ref[0]
    dbc = jnp.dot(den, B1_ref[...], preferred_element_type=F32)
    h = acc / jnp.maximum(dbc, 1e-30) + b_ref[...]
    s1 = jnp.sum(jnp.abs(h), axis=1, keepdims=True)
    h = h / jnp.maximum(s1, 1e-12)
    s2 = jnp.sqrt(jnp.sum(h * h, axis=1, keepdims=True))
    h = h / jnp.maximum(s2, 1e-12)
    out_ref[...] = jnp.maximum(h, 0.0)


def _sds(shape):
    return jax.ShapeDtypeStruct(shape, F32)


_BS_F = pl.BlockSpec((R_SP, D), lambda i, h: (i, 0))
_BS_HROW = pl.BlockSpec((R_SP, DH), lambda i, h: (h * 16 + i, 0))
_BS_A = pl.BlockSpec((R_SP, DH), lambda i, h: (i, 0))
_BS_B = pl.BlockSpec((R_SP, DH), lambda i, h: (i + 16, 0))
_BS_16 = pl.BlockSpec((R_SP, NH), lambda i, h: (i, 0))
_BS_DEN = pl.BlockSpec((1, R_SP, NH), lambda i, h: (0, i, 0))
_BS_WH = pl.BlockSpec((1, D, DH), lambda i, h: (h, 0, 0))
_BS_BH = pl.BlockSpec((1, NH, DH), lambda i, h: (h, 0, 0))


def _fullg(shape):
    return pl.BlockSpec(shape, lambda i, h: (0, 0))


_STACK_OUT = [_BS_HROW, _BS_16, _BS_16, _BS_HROW, _BS_16]
_STACK_SHAPE = [_sds((2 * N_SP, DH)), _sds((N_SP, NH)), _sds((N_SP, NH)),
                _sds((2 * N_SP, DH)), _sds((N_SP, NH))]


def _pre_call(f, W, WAs, WAd, B):
    return pl.pallas_call(
        _pre_body,
        grid=(16, 2),
        in_specs=[_BS_F, _BS_WH, _fullg((D, NH)), _fullg((D, NH)), _BS_BH],
        out_specs=_STACK_OUT,
        out_shape=_STACK_SHAPE,
    )(f, W, WAs, WAd, B)


def _mid_call(acc, den, b, B0, W, WAs, WAd, B1):
    return pl.pallas_call(
        _mid_body,
        grid=(16, 2),
        in_specs=[_BS_A, _BS_B, _BS_DEN,
                  _fullg((1, D)), _fullg((NH, D)), _BS_WH,
                  _fullg((D, NH)), _fullg((D, NH)), _BS_BH],
        out_specs=_STACK_OUT,
        out_shape=_STACK_SHAPE,
    )(acc, acc, den, b, B0, W, WAs, WAd, B1)


def _post_call(acc, den, b, B1):
    return pl.pallas_call(
        _post_body,
        grid=(16, 2),
        in_specs=[_BS_A, _BS_B, _BS_DEN, _fullg((1, D)), _fullg((NH, D))],
        out_specs=_BS_F,
        out_shape=_sds((N_SP, D)),
    )(acc, acc, den, b, B1)


def _lane_splat(v, idx16):
    dn = lax.GatherDimensionNumbers(
        offset_dims=(), collapsed_slice_dims=(0,), start_index_map=(0,))
    return lax.gather(v, idx16.reshape(16, 1), dn, (1,),
                      mode=lax.GatherScatterMode.PROMISE_IN_BOUNDS)


def _make_sc_edge():
    mesh = plsc.VectorSubcoreMesh(core_axis_name="c", subcore_axis_name="s",
                                  num_cores=2)

    buf_set = [
        pltpu.VMEM((SUB, 128), jnp.int32),
        pltpu.VMEM((SUB, 128), jnp.int32),
        pltpu.VMEM((SUB, 128), jnp.int32),
        pltpu.VMEM((SUB, 128), jnp.int32),
        pltpu.VMEM((CH, DH), F32),
        pltpu.VMEM((CH, NH), F32),
        pltpu.VMEM((CH, NH), F32),
        pltpu.VMEM((CH, NH), F32),
    ]

    @functools.partial(
        pl.kernel,
        out_type=[jax.ShapeDtypeStruct((2 * N_SP, DH), F32),
                  jax.ShapeDtypeStruct((2, N_SP, NH), F32)],
        mesh=mesh,
        compiler_params=pltpu.CompilerParams(use_tc_tiling_on_sc=False),
        scratch_types=(buf_set * 3) + [
            pltpu.VMEM((4, 16), jnp.int32),
            pltpu.VMEM_SHARED((N_SP, DH), F32),
            pltpu.VMEM_SHARED((N_SP, NH), F32),
            pltpu.SemaphoreType.DMA,
            pltpu.SemaphoreType.DMA,
            pltpu.SemaphoreType.DMA,
            pltpu.SemaphoreType.DMA,
            pltpu.SemaphoreType.DMA,
            pltpu.SemaphoreType.DMA,
            pltpu.SemaphoreType.DMA,
            pltpu.SemaphoreType.DMA,
            pltpu.SemaphoreType.DMA,
        ],
    )
    def sc_edge(xlh_hbm, asrc_hbm, adst_hbm, src2d_hbm, dst2d_hbm,
                iacch_hbm, iden_hbm, hmap_hbm,
                acc_out, den_out,
                a0, a1, a2, a3, a4, a5, a6, a7,
                b0, b1, b2, b3, b4, b5, b6, b7,
                c0, c1, c2, c3, c4, c5, c6, c7,
                hmap, acc_sp, den_sp,
                g0, g1, g2, s0, s1, s2, i0, i1, i2):
        S = ((a0, a1, a2, a3, a4, a5, a6, a7),
             (b0, b1, b2, b3, b4, b5, b6, b7),
             (c0, c1, c2, c3, c4, c5, c6, c7))
        gsem = (g0, g1, g2)
        ssem = (s0, s1, s2)
        isem = (i0, i1, i2)
        c = lax.axis_index("c")
        s = lax.axis_index("s")
        rbase = s * ROWS_PER_TILE
        rows = pl.ds(rbase, ROWS_PER_TILE)
        hrows = pl.ds(c * N_SP + rbase, ROWS_PER_TILE)

        pltpu.sync_copy(hmap_hbm.at[c], hmap)
        pltpu.sync_copy(iacch_hbm.at[hrows], acc_sp.at[rows])
        pltpu.sync_copy(iden_hbm.at[rows], den_sp.at[rows])
        plsc.subcore_barrier()

        def i_copies(k, st, sem, make):
            f = pltpu.make_async_copy if make else pltpu.async_copy
            row0 = s * ROWS_E + k * SUB
            return [f(src2d_hbm.at[pl.ds(row0, SUB)], st[0], sem),
                    f(dst2d_hbm.at[pl.ds(row0, SUB)], st[1], sem)]

        def fire_i(k, st, sem):
            i_copies(k, st, sem, False)

        def drain_i(st, sem):
            for d in i_copies(0, st, sem, True):
                d.wait()

        def build(st):
            sidx, didx, sidx2, didxd = st[0], st[1], st[2], st[3]
            for g in range(8):
                cs = pl.ds(g * 16, 16)
                for j in range(SUB):
                    sidx2[j, cs] = sidx[j, cs] + c * N_SP
                    didxd[j, cs] = didx[j, cs]

        def g_copies(st, sem, make):
            sidx, didx, sidx2 = st[0], st[1], st[2]
            xg, asg, adg = st[4], st[5], st[6]
            f = pltpu.make_async_copy if make else pltpu.async_copy
            out = []
            for j in range(SUB):
                sl = pl.ds(j * 128, 128)
                out.append(f(asrc_hbm.at[sidx.at[j]], asg.at[sl], sem))
                out.append(f(adst_hbm.at[didx.at[j]], adg.at[sl], sem))
                out.append(f(xlh_hbm.at[sidx2.at[j]], xg.at[sl], sem))
            return out

        def fire_g(st, sem):
            g_copies(st, sem, False)

        def drain_g(st, sem):
            for d in g_copies(st, sem, True):
                d.wait()

        def compute(st):
            xg, asg, adg, pbuf = st[4], st[5], st[6], st[7]

            @plsc.parallel_loop(0, CH, 1, unroll=16)
            def edge(e):
                z = asg[e, :] + adg[e, :]
                p = jnp.exp(jnp.maximum(z, 0.2 * z))
                pbuf[e, :] = p
                for j in range(4):
                    ph = _lane_splat(p, hmap[j, :])
                    cs = pl.ds(j * 16, 16)
                    xg[e, cs] = xg[e, cs] * ph

        def s_copies(st, sem, make):
            didxd, xg, pbuf = st[3], st[4], st[7]
            out = []
            for j in range(SUB):
                sl = pl.ds(j * 128, 128)
                if make:
                    out.append(pltpu.make_async_copy(
                        xg.at[sl], acc_sp.at[didxd.at[j]], sem))
                    out.append(pltpu.make_async_copy(
                        pbuf.at[sl], den_sp.at[didxd.at[j]], sem))
                else:
                    pltpu.async_copy(xg.at[sl], acc_sp.at[didxd.at[j]], sem,
                                     add=True)
                    pltpu.async_copy(pbuf.at[sl], den_sp.at[didxd.at[j]], sem,
                                     add=True)
            return out

        def fire_s(st, sem):
            s_copies(st, sem, False)

        def drain_s(st, sem):
            for d in s_copies(st, sem, True):
                d.wait()

        fire_i(0, S[0], isem[0])
        drain_i(S[0], isem[0])
        build(S[0])
        fire_g(S[0], gsem[0])
        fire_i(1, S[1], isem[1])
        drain_i(S[1], isem[1])
        build(S[1])
        fire_g(S[1], gsem[1])
        fire_i(2, S[2], isem[2])
        drain_g(S[0], gsem[0])
        fire_i(3, S[0], isem[0])
        compute(S[0])
        fire_s(S[0], ssem[0])
        drain_i(S[2], isem[2])
        build(S[2])
        fire_g(S[2], gsem[2])
        drain_g(S[1], gsem[1])
        fire_i(4, S[1], isem[1])
        compute(S[1])
        fire_s(S[1], ssem[1])

        def sub(k, cur, nxt):
            drain_s(S[nxt], ssem[nxt])
            drain_i(S[nxt], isem[nxt])
            build(S[nxt])
            fire_g(S[nxt], gsem[nxt])
            drain_g(S[cur], gsem[cur])
            fire_i(k + 3, S[cur], isem[cur])
            compute(S[cur])
            fire_s(S[cur], ssem[cur])

        def trio(i, carry):
            k = 3 * i + 2
            sub(k, 2, 0)
            sub(k + 1, 0, 1)
            sub(k + 2, 1, 2)
            return carry

        lax.fori_loop(0, (N_CHUNKS - 2) // 3, trio, 0)

        drain_s(S[0], ssem[0])
        drain_s(S[1], ssem[1])
        drain_g(S[2], gsem[2])
        drain_i(S[0], isem[0])
        drain_i(S[1], isem[1])
        plsc.subcore_barrier()

        pltpu.sync_copy(acc_sp.at[rows], acc_out.at[hrows])
        pltpu.sync_copy(den_sp.at[rows], den_out.at[c, rows])

    return sc_edge


_sc_edge = _make_sc_edge()


def _head_mask(H, C):
    M = np.zeros((D, NH), np.float32)
    for h in range(H):
        M[h * C:(h + 1) * C, h] = 1.0
    return jnp.asarray(M)


def _bcast_mat(H, C):
    B = np.zeros((NH, D), np.float32)
    for h in range(H):
        B[h, h * C:(h + 1) * C] = 1.0
    return jnp.asarray(B)


def _prep_edges(ei):
    pad = jnp.full((E_IDX_ROWS * 128 - E,), N, dtype=ei.dtype)
    src = jnp.concatenate([ei[0], pad]).reshape(-1, 128)
    dst = jnp.concatenate([ei[1], pad]).reshape(-1, 128)
    return src, dst


def kernel(x, edge_index0, edge_index1, W0, a_src0, a_dst0, b0,
           W1, a_src1, a_dst1, b1):
    As0 = a_src0.reshape(D, 1) * _head_mask(4, 32)
    Ad0 = a_dst0.reshape(D, 1) * _head_mask(4, 32)
    B0 = _bcast_mat(4, 32)
    As1 = a_src1.reshape(D, 1) * _head_mask(1, 128)
    Ad1 = a_dst1.reshape(D, 1) * _head_mask(1, 128)
    B1 = _bcast_mat(1, 128)
    WAs0, WAd0 = W0 @ As0, W0 @ Ad0
    WAs1, WAd1 = W1 @ As1, W1 @ Ad1
    W0s = jnp.stack([W0[:, :DH], W0[:, DH:]])
    W1s = jnp.stack([W1[:, :DH], W1[:, DH:]])
    B0s = jnp.stack([B0[:, :DH], B0[:, DH:]])
    B1s = jnp.stack([B1[:, :DH], B1[:, DH:]])
    s0, d0 = _prep_edges(edge_index0)
    s1, d1 = _prep_edges(edge_index1)
    b0r = b0.reshape(1, D)
    b1r = b1.reshape(1, D)
    x_pad = jnp.concatenate([x, jnp.zeros((N_SP - N, D), F32)], axis=0)

    hmap0 = jnp.asarray(
        np.repeat(np.arange(4, dtype=np.int32), 2).reshape(2, 4)
        .repeat(16, axis=1).reshape(2, 4, 16))
    hmap1 = jnp.zeros((2, 4, 16), jnp.int32)

    xlh0, at0, dt0, iacch0, iden0 = _pre_call(x_pad, W0s, WAs0, WAd0, B0s)
    acc0, den0 = _sc_edge(xlh0, at0, dt0, s0, d0, iacch0, iden0, hmap0)
    xlh1, at1, dt1, iacch1, iden1 = _mid_call(
        acc0, den0, b0r, B0, W1s, WAs1, WAd1, B1s)
    acc1, den1 = _sc_edge(xlh1, at1, dt1, s1, d1, iacch1, iden1, hmap1)
    out = _post_call(acc1, den1, b1r, B1)
    return out[:N]


# speedup vs baseline: 1.0121x; 1.0121x over previous
import functools

import jax
import jax.numpy as jnp
import numpy as np
from jax import lax
from jax.experimental import pallas as pl
from jax.experimental.pallas import tpu as pltpu
from jax.experimental.pallas import tpu_sc as plsc

N = 10000
D = 128
DH = 64
NH = 16
N_SP = 10112
ROWS_PER_TILE = N_SP // 16
E = 320000
CH = 128
SUB = CH // 128
N_CHUNKS = 161
EP_T = N_CHUNKS * CH
E_PAD = EP_T * 16
ROWS_E = EP_T // 128
E_IDX_ROWS = E_PAD // 128 + 8
R_BLK = 400
F32 = jnp.float32


R_SP = N_SP // 16


def _prep_tail(f, Wh_ref, WAs_ref, WAd_ref, Bh_ref):
    xlh = jnp.dot(f, Wh_ref[0], preferred_element_type=F32)
    at = jnp.dot(f, WAs_ref[...], preferred_element_type=F32)
    dt = jnp.dot(f, WAd_ref[...], preferred_element_type=F32)
    z = at + dt
    pd = jnp.exp(jnp.maximum(z, 0.2 * z))
    iacch = xlh * jnp.dot(pd, Bh_ref[0], preferred_element_type=F32)
    return xlh, at, dt, iacch, pd


def _pre_body(f_ref, Wh_ref, WAs_ref, WAd_ref, Bh_ref,
              xlh_ref, at_ref, dt_ref, iacch_ref, iden_ref):
    xlh, at, dt, iacch, pd = _prep_tail(f_ref[...], Wh_ref, WAs_ref,
                                        WAd_ref, Bh_ref)
    xlh_ref[...] = xlh
    at_ref[...] = at
    dt_ref[...] = dt
    iacch_ref[...] = iacch
    iden_ref[...] = pd


def _mid_body(acca_ref, accb_ref, den_ref, b_ref, B0_ref,
              Wh_ref, WAs_ref, WAd_ref, Bh_ref,
              xlh_ref, at_ref, dt_ref, iacch_ref, iden_ref):
    acc = jnp.concatenate([acca_ref[...], accb_ref[...]], axis=1)
    den = den_ref[0]
    dbc = jnp.dot(den, B0_ref[...], preferred_element_type=F32)
    h = acc / jnp.maximum(dbc, 1e-30) + b_ref[...]
    h = jnp.maximum(h, 0.0)
    s = jnp.sum(jnp.abs(h), axis=1, keepdims=True)
    h = h / jnp.maximum(s, 1e-12)
    xlh, at, dt, iacch, pd = _prep_tail(h, Wh_ref, WAs_ref, WAd_ref, Bh_ref)
    xlh_ref[...] = xlh
    at_ref[...] = at
    dt_ref[...] = dt
    iacch_ref[...] = iacch
    iden_ref[...] = pd


def _post_body(acca_ref, accb_ref, den_ref, b_ref, B1_ref, out_ref):
    acc = jnp.concatenate([acca_ref[...], accb_ref[...]], axis=1)
    den = den_ref[0]
    dbc = jnp.dot(den, B1_ref[...], preferred_element_type=F32)
    h = acc / jnp.maximum(dbc, 1e-30) + b_ref[...]
    s1 = jnp.sum(jnp.abs(h), axis=1, keepdims=True)
    h = h / jnp.maximum(s1, 1e-12)
    s2 = jnp.sqrt(jnp.sum(h * h, axis=1, keepdims=True))
    h = h / jnp.maximum(s2, 1e-12)
    out_ref[...] = jnp.maximum(h, 0.0)


def _sds(shape):
    return jax.ShapeDtypeStruct(shape, F32)


_BS_F = pl.BlockSpec((R_SP, D), lambda i, h: (i, 0))
_BS_HROW = pl.BlockSpec((R_SP, DH), lambda i, h: (h * 16 + i, 0))
_BS_A = pl.BlockSpec((R_SP, DH), lambda i, h: (i, 0))
_BS_B = pl.BlockSpec((R_SP, DH), lambda i, h: (i + 16, 0))
_BS_16 = pl.BlockSpec((R_SP, NH), lambda i, h: (i, 0))
_BS_DEN = pl.BlockSpec((1, R_SP, NH), lambda i, h: (0, i, 0))
_BS_WH = pl.BlockSpec((1, D, DH), lambda i, h: (h, 0, 0))
_BS_BH = pl.BlockSpec((1, NH, DH), lambda i, h: (h, 0, 0))


def _fullg(shape):
    return pl.BlockSpec(shape, lambda i, h: (0, 0))


_STACK_OUT = [_BS_HROW, _BS_16, _BS_16, _BS_HROW, _BS_16]
_STACK_SHAPE = [_sds((2 * N_SP, DH)), _sds((N_SP, NH)), _sds((N_SP, NH)),
                _sds((2 * N_SP, DH)), _sds((N_SP, NH))]


def _pre_call(f, W, WAs, WAd, B):
    return pl.pallas_call(
        _pre_body,
        grid=(16, 2),
        in_specs=[_BS_F, _BS_WH, _fullg((D, NH)), _fullg((D, NH)), _BS_BH],
        out_specs=_STACK_OUT,
        out_shape=_STACK_SHAPE,
    )(f, W, WAs, WAd, B)


def _mid_call(acc, den, b, B0, W, WAs, WAd, B1):
    return pl.pallas_call(
        _mid_body,
        grid=(16, 2),
        in_specs=[_BS_A, _BS_B, _BS_DEN,
                  _fullg((1, D)), _fullg((NH, D)), _BS_WH,
                  _fullg((D, NH)), _fullg((D, NH)), _BS_BH],
        out_specs=_STACK_OUT,
        out_shape=_STACK_SHAPE,
    )(acc, acc, den, b, B0, W, WAs, WAd, B1)


def _post_call(acc, den, b, B1):
    return pl.pallas_call(
        _post_body,
        grid=(16, 2),
        in_specs=[_BS_A, _BS_B, _BS_DEN, _fullg((1, D)), _fullg((NH, D))],
        out_specs=_BS_F,
        out_shape=_sds((N_SP, D)),
    )(acc, acc, den, b, B1)


def _lane_splat(v, idx16):
    dn = lax.GatherDimensionNumbers(
        offset_dims=(), collapsed_slice_dims=(0,), start_index_map=(0,))
    return lax.gather(v, idx16.reshape(16, 1), dn, (1,),
                      mode=lax.GatherScatterMode.PROMISE_IN_BOUNDS)


def _make_sc_edge():
    mesh = plsc.VectorSubcoreMesh(core_axis_name="c", subcore_axis_name="s",
                                  num_cores=2)

    buf_set = [
        pltpu.VMEM((SUB, 128), jnp.int32),
        pltpu.VMEM((SUB, 128), jnp.int32),
        pltpu.VMEM((SUB, 128), jnp.int32),
        pltpu.VMEM((SUB, 128), jnp.int32),
        pltpu.VMEM((CH, DH), F32),
        pltpu.VMEM((CH, NH), F32),
        pltpu.VMEM((CH, NH), F32),
        pltpu.VMEM((CH, NH), F32),
    ]

    @functools.partial(
        pl.kernel,
        out_type=[jax.ShapeDtypeStruct((2 * N_SP, DH), F32),
                  jax.ShapeDtypeStruct((2, N_SP, NH), F32)],
        mesh=mesh,
        compiler_params=pltpu.CompilerParams(use_tc_tiling_on_sc=False),
        scratch_types=(buf_set * 3) + [
            pltpu.VMEM((4, 16), jnp.int32),
            pltpu.VMEM_SHARED((N_SP, DH), F32),
            pltpu.VMEM_SHARED((N_SP, NH), F32),
            pltpu.SemaphoreType.DMA,
            pltpu.SemaphoreType.DMA,
            pltpu.SemaphoreType.DMA,
            pltpu.SemaphoreType.DMA,
            pltpu.SemaphoreType.DMA,
            pltpu.SemaphoreType.DMA,
            pltpu.SemaphoreType.DMA,
            pltpu.SemaphoreType.DMA,
            pltpu.SemaphoreType.DMA,
        ],
    )
    def sc_edge(xlh_hbm, asrc_hbm, adst_hbm, src2d_hbm, dst2d_hbm,
                iacch_hbm, iden_hbm, hmap_hbm,
                acc_out, den_out,
                a0, a1, a2, a3, a4, a5, a6, a7,
                b0, b1, b2, b3, b4, b5, b6, b7,
                c0, c1, c2, c3, c4, c5, c6, c7,
                hmap, acc_sp, den_sp,
                g0, g1, g2, s0, s1, s2, i0, i1, i2):
        S = ((a0, a1, a2, a3, a4, a5, a6, a7),
             (b0, b1, b2, b3, b4, b5, b6, b7),
             (c0, c1, c2, c3, c4, c5, c6, c7))
        gsem = (g0, g1, g2)
        ssem = (s0, s1, s2)
        isem = (i0, i1, i2)
        c = lax.axis_index("c")
        s = lax.axis_index("s")
        rbase = s * ROWS_PER_TILE
        rows = pl.ds(rbase, ROWS_PER_TILE)
        hrows = pl.ds(c * N_SP + rbase, ROWS_PER_TILE)

        pltpu.sync_copy(hmap_hbm.at[c], hmap)
        pltpu.sync_copy(iacch_hbm.at[hrows], acc_sp.at[rows])
        pltpu.sync_copy(iden_hbm.at[rows], den_sp.at[rows])
        plsc.subcore_barrier()

        def i_copies(k, st, sem, make):
            f = pltpu.make_async_copy if make else pltpu.async_copy
            row0 = s * ROWS_E + k * SUB
            return [f(src2d_hbm.at[pl.ds(row0, SUB)], st[0], sem),
                    f(dst2d_hbm.at[pl.ds(row0, SUB)], st[1], sem)]

        def fire_i(k, st, sem):
            i_copies(k, st, sem, False)

        def drain_i(st, sem):
            for d in i_copies(0, st, sem, True):
                d.wait()

        def build(st):
            sidx, didx, sidx2, didxd = st[0], st[1], st[2], st[3]
            for g in range(8):
                cs = pl.ds(g * 16, 16)
                for j in range(SUB):
                    sidx2[j, cs] = sidx[j, cs] + c * N_SP
                    didxd[j, cs] = didx[j, cs]

        def g_copies(st, sem, make):
            sidx, didx, sidx2 = st[0], st[1], st[2]
            xg, asg, adg = st[4], st[5], st[6]
            f = pltpu.make_async_copy if make else pltpu.async_copy
            out = []
            for j in range(SUB):
                sl = pl.ds(j * 128, 128)
                out.append(f(asrc_hbm.at[sidx.at[j]], asg.at[sl], sem))
                out.append(f(adst_hbm.at[didx.at[j]], adg.at[sl], sem))
                out.append(f(xlh_hbm.at[sidx2.at[j]], xg.at[sl], sem))
            return out

        def fire_g(st, sem):
            g_copies(st, sem, False)

        def drain_g(st, sem):
            for d in g_copies(st, sem, True):
                d.wait()

        def compute(st):
            xg, asg, adg, pbuf = st[4], st[5], st[6], st[7]

            @plsc.parallel_loop(0, CH, 1, unroll=8)
            def edge(e):
                z = asg[e, :] + adg[e, :]
                p = jnp.exp(jnp.maximum(z, 0.2 * z))
                pbuf[e, :] = p
                for j in range(4):
                    ph = _lane_splat(p, hmap[j, :])
                    cs = pl.ds(j * 16, 16)
                    xg[e, cs] = xg[e, cs] * ph

        def s_copies(st, sem, make):
            didxd, xg, pbuf = st[3], st[4], st[7]
            out = []
            for j in range(SUB):
                sl = pl.ds(j * 128, 128)
                if make:
                    out.append(pltpu.make_async_copy(
                        xg.at[sl], acc_sp.at[didxd.at[j]], sem))
                    out.append(pltpu.make_async_copy(
                        pbuf.at[sl], den_sp.at[didxd.at[j]], sem))
                else:
                    pltpu.async_copy(xg.at[sl], acc_sp.at[didxd.at[j]], sem,
                                     add=True)
                    pltpu.async_copy(pbuf.at[sl], den_sp.at[didxd.at[j]], sem,
                                     add=True)
            return out

        def fire_s(st, sem):
            s_copies(st, sem, False)

        def drain_s(st, sem):
            for d in s_copies(st, sem, True):
                d.wait()

        fire_i(0, S[0], isem[0])
        drain_i(S[0], isem[0])
        build(S[0])
        fire_g(S[0], gsem[0])
        fire_i(1, S[1], isem[1])
        drain_i(S[1], isem[1])
        build(S[1])
        fire_g(S[1], gsem[1])
        fire_i(2, S[2], isem[2])
        drain_g(S[0], gsem[0])
        fire_i(3, S[0], isem[0])
        compute(S[0])
        fire_s(S[0], ssem[0])
        drain_i(S[2], isem[2])
        build(S[2])
        fire_g(S[2], gsem[2])
        drain_g(S[1], gsem[1])
        fire_i(4, S[1], isem[1])
        compute(S[1])
        fire_s(S[1], ssem[1])

        def sub(k, cur, nxt):
            drain_s(S[nxt], ssem[nxt])
            drain_i(S[nxt], isem[nxt])
            build(S[nxt])
            fire_g(S[nxt], gsem[nxt])
            drain_g(S[cur], gsem[cur])
            fire_i(k + 3, S[cur], isem[cur])
            compute(S[cur])
            fire_s(S[cur], ssem[cur])

        def trio(i, carry):
            k = 3 * i + 2
            sub(k, 2, 0)
            sub(k + 1, 0, 1)
            sub(k + 2, 1, 2)
            return carry

        lax.fori_loop(0, (N_CHUNKS - 2) // 3, trio, 0)

        drain_s(S[0], ssem[0])
        drain_s(S[1], ssem[1])
        drain_g(S[2], gsem[2])
        drain_i(S[0], isem[0])
        drain_i(S[1], isem[1])
        plsc.subcore_barrier()

        pltpu.sync_copy(acc_sp.at[rows], acc_out.at[hrows])
        pltpu.sync_copy(den_sp.at[rows], den_out.at[c, rows])

    return sc_edge


_sc_edge = _make_sc_edge()


def _head_mask(H, C):
    M = np.zeros((D, NH), np.float32)
    for h in range(H):
        M[h * C:(h + 1) * C, h] = 1.0
    return jnp.asarray(M)


def _bcast_mat(H, C):
    B = np.zeros((NH, D), np.float32)
    for h in range(H):
        B[h, h * C:(h + 1) * C] = 1.0
    return jnp.asarray(B)


def _prep_edges(ei):
    pad = jnp.full((E_IDX_ROWS * 128 - E,), N, dtype=ei.dtype)
    src = jnp.concatenate([ei[0], pad]).reshape(-1, 128)
    dst = jnp.concatenate([ei[1], pad]).reshape(-1, 128)
    return src, dst


def kernel(x, edge_index0, edge_index1, W0, a_src0, a_dst0, b0,
           W1, a_src1, a_dst1, b1):
    As0 = a_src0.reshape(D, 1) * _head_mask(4, 32)
    Ad0 = a_dst0.reshape(D, 1) * _head_mask(4, 32)
    B0 = _bcast_mat(4, 32)
    As1 = a_src1.reshape(D, 1) * _head_mask(1, 128)
    Ad1 = a_dst1.reshape(D, 1) * _head_mask(1, 128)
    B1 = _bcast_mat(1, 128)
    WAs0, WAd0 = W0 @ As0, W0 @ Ad0
    WAs1, WAd1 = W1 @ As1, W1 @ Ad1
    W0s = jnp.stack([W0[:, :DH], W0[:, DH:]])
    W1s = jnp.stack([W1[:, :DH], W1[:, DH:]])
    B0s = jnp.stack([B0[:, :DH], B0[:, DH:]])
    B1s = jnp.stack([B1[:, :DH], B1[:, DH:]])
    s0, d0 = _prep_edges(edge_index0)
    s1, d1 = _prep_edges(edge_index1)
    b0r = b0.reshape(1, D)
    b1r = b1.reshape(1, D)
    x_pad = jnp.concatenate([x, jnp.zeros((N_SP - N, D), F32)], axis=0)

    hmap0 = jnp.asarray(
        np.repeat(np.arange(4, dtype=np.int32), 2).reshape(2, 4)
        .repeat(16, axis=1).reshape(2, 4, 16))
    hmap1 = jnp.zeros((2, 4, 16), jnp.int32)

    xlh0, at0, dt0, iacch0, iden0 = _pre_call(x_pad, W0s, WAs0, WAd0, B0s)
    acc0, den0 = _sc_edge(xlh0, at0, dt0, s0, d0, iacch0, iden0, hmap0)
    xlh1, at1, dt1, iacch1, iden1 = _mid_call(
        acc0, den0, b0r, B0, W1s, WAs1, WAd1, B1s)
    acc1, den1 = _sc_edge(xlh1, at1, dt1, s1, d1, iacch1, iden1, hmap1)
    out = _post_call(acc1, den1, b1r, B1)
    return out[:N]
